# Initial kernel scaffold; baseline (speedup 1.0000x reference)
import jax
import jax.numpy as jnp
from jax.experimental import pallas as pl


def kernel(x, edge_index, batch, W_lin, b_lin, W1, b1, W2, b2, W3, b3, W_emb, b_emb, W_pred, b_pred):
    raise NotImplementedError("write your pallas kernel here")


# speedup vs baseline: 18.2796x; 18.2796x over previous
import functools

import jax
import jax.numpy as jnp
from jax import lax
from jax.experimental import pallas as pl
from jax.experimental.pallas import tpu as pltpu
from jax.experimental.pallas import tpu_sc as plsc

N = 50000
E = 800000
D_IN = 128
DH = 64
NG = 256
HALF = DH // 2

NP = 51200
NDUM = NP - N
NPOOL = 1280

C = 512
KC = 98
EPC = 16 * KC * C
KD = 49
EPD = 32 * KD * C
KP = 7
EPP = 16 * KP * C

BN = 1000


def _make_edge_scatter(num_rows, k_chunks, split_edges_over_cores, do_gather):
  mesh = plsc.VectorSubcoreMesh(core_axis_name="c", subcore_axis_name="s")
  zr = num_rows // 16

  scratch = []
  if do_gather:
    scratch.append(pltpu.VMEM((C,), jnp.int32))
  scratch += [
      pltpu.VMEM((C,), jnp.int32),
      pltpu.VMEM((C, HALF), jnp.float32),
      pltpu.VMEM_SHARED((num_rows, HALF), jnp.float32),
      pltpu.SemaphoreType.DMA,
  ]

  def body(*refs):
    if do_gather:
      table, src, dst, out, sidx, didx, rows, acc, sem = refs
    else:
      dst, out, didx, rows, acc, sem = refs
    c = lax.axis_index("c")
    s = lax.axis_index("s")

    def fill_rows(val):
      v16 = jnp.full((16,), val, jnp.float32)
      def fb(r, carry):
        for j in range(HALF // 16):
          rows[r, pl.ds(j * 16, 16)] = v16
        return carry
      lax.fori_loop(0, C, fb, 0)

    fill_rows(0.0)
    row0 = s * zr
    off = 0
    while off < zr:
      step = min(C, zr - off)
      pltpu.sync_copy(rows.at[pl.ds(0, step)], acc.at[pl.ds(row0 + off, step)])
      off += step
    plsc.subcore_barrier()

    if not do_gather:
      fill_rows(1.0)

    if split_edges_over_cores:
      base0 = (c * 16 + s) * (k_chunks * C)
    else:
      base0 = s * (k_chunks * C)
    coff = c * N

    def edge_body(k, carry):
      base = base0 + k * C
      pltpu.sync_copy(dst.at[pl.ds(base, C)], didx)
      if do_gather:
        pltpu.sync_copy(src.at[pl.ds(base, C)], sidx)
        for j in range(C // 16):
          sl = pl.ds(j * 16, 16)
          sidx[sl] = sidx[sl] + coff
        pltpu.async_copy(table.at[sidx], rows, sem).wait()
      pltpu.sync_copy(rows, acc.at[didx], add=True)
      return carry

    lax.fori_loop(0, k_chunks, edge_body, 0)
    plsc.subcore_barrier()

    out_base = c * num_rows + s * zr
    off = 0
    while off < zr:
      step = min(C, zr - off)
      pltpu.sync_copy(acc.at[pl.ds(row0 + off, step)], rows.at[pl.ds(0, step)])
      pltpu.sync_copy(rows.at[pl.ds(0, step)], out.at[pl.ds(out_base + off, step)])
      off += step

  return pl.kernel(
      body,
      out_type=jax.ShapeDtypeStruct((2 * num_rows, HALF), jnp.float32),
      mesh=mesh,
      scratch_types=scratch,
      compiler_params=pltpu.CompilerParams(use_tc_tiling_on_sc=False),
  )


_deg_sc = _make_edge_scatter(NP, KD, split_edges_over_cores=True, do_gather=False)
_conv_sc = _make_edge_scatter(NP, KC, split_edges_over_cores=False, do_gather=True)
_pool_sc = _make_edge_scatter(NPOOL, KP, split_edges_over_cores=False, do_gather=True)


def _deg_cols(deg_ref):
  degf = deg_ref[0][:, :1] + deg_ref[1][:, :1] + 1.0
  return lax.rsqrt(degf), 1.0 / degf


def _split_store(tab_ref, t):
  tab_ref[0] = t[:, :HALF]
  tab_ref[1] = t[:, HALF:]


def _tc0_body(x_ref, deg_ref, wlin_ref, blin_ref, w1_ref, tab_ref, self_ref):
  dis, invd = _deg_cols(deg_ref)
  h0 = jnp.dot(x_ref[...], wlin_ref[...], preferred_element_type=jnp.float32)
  h0 = h0 + blin_ref[...]
  t1 = jnp.dot(h0, w1_ref[...], preferred_element_type=jnp.float32)
  self_ref[...] = t1 * invd
  _split_store(tab_ref, t1 * dis)


def _tc_mid_body(acc_ref, selfin_ref, deg_ref, b_ref, w_ref, tab_ref,
                 selfout_ref):
  dis, invd = _deg_cols(deg_ref)
  m = jnp.concatenate([acc_ref[0], acc_ref[1]], axis=1)
  conv = jnp.maximum(dis * m + selfin_ref[...] + b_ref[...], 0.0)
  t = jnp.dot(conv, w_ref[...], preferred_element_type=jnp.float32)
  selfout_ref[...] = t * invd
  _split_store(tab_ref, t * dis)


def _tc_emb_body(acc_ref, selfin_ref, deg_ref, b_ref, wemb_ref, bemb_ref,
                 tab_ref):
  dis, _ = _deg_cols(deg_ref)
  m = jnp.concatenate([acc_ref[0], acc_ref[1]], axis=1)
  conv = jnp.maximum(dis * m + selfin_ref[...] + b_ref[...], 0.0)
  he = jnp.dot(conv, wemb_ref[...], preferred_element_type=jnp.float32)
  he = jnp.maximum(he + bemb_ref[...], 0.0)
  _split_store(tab_ref, he)


def _tc_pred_body(accp_ref, wpred_ref, bpred_ref, out_ref):
  g = jnp.concatenate([accp_ref[0], accp_ref[1]], axis=1)
  out_ref[...] = (
      jnp.dot(g, wpred_ref[...], preferred_element_type=jnp.float32)
      + bpred_ref[...]
  )


def _full(shape):
  return pl.BlockSpec(shape, lambda i: tuple(0 for _ in shape))

_GRID = N // BN
_acc_spec = pl.BlockSpec((2, BN, HALF), lambda i: (0, i, 0))
_deg_spec = pl.BlockSpec((2, BN, HALF), lambda i: (0, i, 0))
_self_spec = pl.BlockSpec((BN, DH), lambda i: (i, 0))
_tab_spec = pl.BlockSpec((2, BN, HALF), lambda i: (0, i, 0))

_tab_shape = jax.ShapeDtypeStruct((2, N, HALF), jnp.float32)
_self_shape = jax.ShapeDtypeStruct((N, DH), jnp.float32)

_tc0 = pl.pallas_call(
    _tc0_body,
    grid=(_GRID,),
    in_specs=[
        pl.BlockSpec((BN, D_IN), lambda i: (i, 0)),
        _deg_spec,
        _full((D_IN, DH)),
        _full((1, DH)),
        _full((DH, DH)),
    ],
    out_specs=[_tab_spec, _self_spec],
    out_shape=[_tab_shape, _self_shape],
)

_tc_mid = pl.pallas_call(
    _tc_mid_body,
    grid=(_GRID,),
    in_specs=[_acc_spec, _self_spec, _deg_spec, _full((1, DH)),
              _full((DH, DH))],
    out_specs=[_tab_spec, _self_spec],
    out_shape=[_tab_shape, _self_shape],
)

_tc_emb = pl.pallas_call(
    _tc_emb_body,
    grid=(_GRID,),
    in_specs=[_acc_spec, _self_spec, _deg_spec, _full((1, DH)),
              _full((DH, DH)), _full((1, DH))],
    out_specs=[_tab_spec],
    out_shape=[_tab_shape],
)

_tc_pred = pl.pallas_call(
    _tc_pred_body,
    grid=(1,),
    in_specs=[
        pl.BlockSpec((2, NG, HALF), lambda i: (0, 0, 0)),
        _full((DH, 1)),
        _full((1, 1)),
    ],
    out_specs=[pl.BlockSpec((NG, 1), lambda i: (0, 0))],
    out_shape=[jax.ShapeDtypeStruct((NG, 1), jnp.float32)],
)


def kernel(x, edge_index, batch, W_lin, b_lin, W1, b1, W2, b2, W3, b3,
           W_emb, b_emb, W_pred, b_pred):
  src = edge_index[0]
  dst = edge_index[1]
  i32 = jnp.int32

  pad = EPC - E
  src_c = jnp.concatenate([src, jnp.arange(pad, dtype=i32) % N])
  dst_c = jnp.concatenate([dst, N + jnp.arange(pad, dtype=i32) % NDUM])
  pad = EPD - E
  dst_d = jnp.concatenate([dst, N + jnp.arange(pad, dtype=i32) % NDUM])
  pad = EPP - N
  src_p = jnp.concatenate(
      [jnp.arange(N, dtype=i32), jnp.arange(pad, dtype=i32) % N])
  dst_p = jnp.concatenate(
      [batch, NG + jnp.arange(pad, dtype=i32) % (NPOOL - NG)])

  b_lin2 = b_lin.reshape(1, DH)
  b1_2, b2_2, b3_2 = b1.reshape(1, DH), b2.reshape(1, DH), b3.reshape(1, DH)
  b_emb2 = b_emb.reshape(1, DH)
  b_pred2 = b_pred.reshape(1, 1)

  deg2 = _deg_sc(dst_d).reshape(2, NP, HALF)
  tab1, self1 = _tc0(x, deg2, W_lin, b_lin2, W1)
  acc1 = _conv_sc(tab1.reshape(2 * N, HALF), src_c, dst_c).reshape(2, NP, HALF)
  tab2, self2 = _tc_mid(acc1, self1, deg2, b1_2, W2)
  acc2 = _conv_sc(tab2.reshape(2 * N, HALF), src_c, dst_c).reshape(2, NP, HALF)
  tab3, self3 = _tc_mid(acc2, self2, deg2, b2_2, W3)
  acc3 = _conv_sc(tab3.reshape(2 * N, HALF), src_c, dst_c).reshape(2, NP, HALF)
  tabe, = _tc_emb(acc3, self3, deg2, b3_2, W_emb, b_emb2)
  accp = _pool_sc(tabe.reshape(2 * N, HALF), src_p, dst_p)
  accp = accp.reshape(2, NPOOL, HALF)
  out, = _tc_pred(accp, W_pred, b_pred2)
  return out


# speedup vs baseline: 22.8257x; 1.2487x over previous
import functools

import jax
import jax.numpy as jnp
from jax import lax
from jax.experimental import pallas as pl
from jax.experimental.pallas import tpu as pltpu
from jax.experimental.pallas import tpu_sc as plsc

N = 50000
E = 800000
D_IN = 128
DH = 64
NG = 256
HALF = DH // 2

NP = 50176
NDUM = NP - N
NPOOL = 1280

C = 448
KC = 112
EPC = 16 * KC * C
KD = 56
EPD = 32 * KD * C
KP = 8
EPP = 16 * KP * C

BN = 1000


def _make_edge_scatter(num_rows, k_chunks, split_edges_over_cores, do_gather):
  mesh = plsc.VectorSubcoreMesh(core_axis_name="c", subcore_axis_name="s")
  zr = num_rows // 16

  assert k_chunks % 2 == 0
  scratch = []
  if do_gather:
    scratch += [pltpu.VMEM((C,), jnp.int32)] * 2
  scratch += [pltpu.VMEM((C,), jnp.int32)] * 2
  nrows = 2 if do_gather else 1
  scratch += [pltpu.VMEM((C, HALF), jnp.float32)] * nrows
  scratch += [
      pltpu.VMEM_SHARED((num_rows, HALF), jnp.float32),
  ]
  scratch += [pltpu.SemaphoreType.DMA] * 4

  def body(*refs):
    if do_gather:
      (table, src, dst, out, sixa, sixb, dixa, dixb, rwa, rwb, acc,
       sga, sgb, ssa, ssb) = refs
    else:
      dst, out, dixa, dixb, rwa, acc, sga, sgb, ssa, ssb = refs
      rwb = rwa
    c = lax.axis_index("c")
    s = lax.axis_index("s")

    def fill_rows(val):
      v16 = jnp.full((16,), val, jnp.float32)
      def fb(r, carry):
        for j in range(HALF // 16):
          rwa[r, pl.ds(j * 16, 16)] = v16
        return carry
      lax.fori_loop(0, C, fb, 0)

    fill_rows(0.0)
    row0 = s * zr
    off = 0
    while off < zr:
      step = min(C, zr - off)
      pltpu.sync_copy(rwa.at[pl.ds(0, step)], acc.at[pl.ds(row0 + off, step)])
      off += step
    plsc.subcore_barrier()

    if not do_gather:
      fill_rows(1.0)

    if split_edges_over_cores:
      base0 = (c * 16 + s) * (k_chunks * C)
    else:
      base0 = s * (k_chunks * C)
    coff = c * N

    def load_idx(k, six, dix):
      base = base0 + k * C
      pltpu.sync_copy(dst.at[pl.ds(base, C)], dix)
      if do_gather:
        pltpu.sync_copy(src.at[pl.ds(base, C)], six)
        for j in range(C // 16):
          sl = pl.ds(j * 16, 16)
          six[sl] = six[sl] + coff

    def gat(six, rw, sem):
      return pltpu.make_async_copy(table.at[six], rw, sem)

    def sca(rw, dix, sem):
      return pltpu.make_async_copy(rw, acc.at[dix], sem)

    half = k_chunks // 2
    if do_gather:
      load_idx(0, sixa, dixa)
      gat(sixa, rwa, sga).start()

      def pair(m, carry):
        load_idx(2 * m + 1, sixb, dixb)
        gat(sixb, rwb, sgb).start()
        gat(sixa, rwa, sga).wait()
        sca(rwa, dixa, ssa).start(add=True)
        gat(sixb, rwb, sgb).wait()
        sca(rwb, dixb, ssb).start(add=True)
        sca(rwa, dixa, ssa).wait()
        load_idx(2 * m + 2, sixa, dixa)
        gat(sixa, rwa, sga).start()
        sca(rwb, dixb, ssb).wait()
        return carry

      lax.fori_loop(0, half - 1, pair, 0)
      load_idx(k_chunks - 1, sixb, dixb)
      gat(sixb, rwb, sgb).start()
      gat(sixa, rwa, sga).wait()
      sca(rwa, dixa, ssa).start(add=True)
      gat(sixb, rwb, sgb).wait()
      sca(rwb, dixb, ssb).start(add=True)
      sca(rwa, dixa, ssa).wait()
      sca(rwb, dixb, ssb).wait()
    else:
      load_idx(0, None, dixa)

      def pair_d(m, carry):
        sca(rwa, dixa, ssa).start(add=True)
        load_idx(2 * m + 1, None, dixb)
        sca(rwb, dixb, ssb).start(add=True)
        sca(rwa, dixa, ssa).wait()
        load_idx(2 * m + 2, None, dixa)
        sca(rwb, dixb, ssb).wait()
        return carry

      lax.fori_loop(0, half - 1, pair_d, 0)
      sca(rwa, dixa, ssa).start(add=True)
      load_idx(k_chunks - 1, None, dixb)
      sca(rwb, dixb, ssb).start(add=True)
      sca(rwa, dixa, ssa).wait()
      sca(rwb, dixb, ssb).wait()

    plsc.subcore_barrier()

    out_base = c * num_rows + s * zr
    off = 0
    while off < zr:
      step = min(C, zr - off)
      pltpu.sync_copy(acc.at[pl.ds(row0 + off, step)], rwa.at[pl.ds(0, step)])
      pltpu.sync_copy(rwa.at[pl.ds(0, step)], out.at[pl.ds(out_base + off, step)])
      off += step

  return pl.kernel(
      body,
      out_type=jax.ShapeDtypeStruct((2 * num_rows, HALF), jnp.float32),
      mesh=mesh,
      scratch_types=scratch,
      compiler_params=pltpu.CompilerParams(use_tc_tiling_on_sc=False),
  )


_deg_sc = _make_edge_scatter(NP, KD, split_edges_over_cores=True, do_gather=False)
_conv_sc = _make_edge_scatter(NP, KC, split_edges_over_cores=False, do_gather=True)
_pool_sc = _make_edge_scatter(NPOOL, KP, split_edges_over_cores=False, do_gather=True)


def _deg_cols(deg_ref):
  degf = deg_ref[0][:, :1] + deg_ref[1][:, :1] + 1.0
  return lax.rsqrt(degf), 1.0 / degf


def _split_store(tab_ref, t):
  tab_ref[0] = t[:, :HALF]
  tab_ref[1] = t[:, HALF:]


def _tc0_body(x_ref, deg_ref, wlin_ref, blin_ref, w1_ref, tab_ref, self_ref):
  dis, invd = _deg_cols(deg_ref)
  h0 = jnp.dot(x_ref[...], wlin_ref[...], preferred_element_type=jnp.float32)
  h0 = h0 + blin_ref[...]
  t1 = jnp.dot(h0, w1_ref[...], preferred_element_type=jnp.float32)
  self_ref[...] = t1 * invd
  _split_store(tab_ref, t1 * dis)


def _tc_mid_body(acc_ref, selfin_ref, deg_ref, b_ref, w_ref, tab_ref,
                 selfout_ref):
  dis, invd = _deg_cols(deg_ref)
  m = jnp.concatenate([acc_ref[0], acc_ref[1]], axis=1)
  conv = jnp.maximum(dis * m + selfin_ref[...] + b_ref[...], 0.0)
  t = jnp.dot(conv, w_ref[...], preferred_element_type=jnp.float32)
  selfout_ref[...] = t * invd
  _split_store(tab_ref, t * dis)


def _tc_emb_body(acc_ref, selfin_ref, deg_ref, b_ref, wemb_ref, bemb_ref,
                 tab_ref):
  dis, _ = _deg_cols(deg_ref)
  m = jnp.concatenate([acc_ref[0], acc_ref[1]], axis=1)
  conv = jnp.maximum(dis * m + selfin_ref[...] + b_ref[...], 0.0)
  he = jnp.dot(conv, wemb_ref[...], preferred_element_type=jnp.float32)
  he = jnp.maximum(he + bemb_ref[...], 0.0)
  _split_store(tab_ref, he)


def _tc_pred_body(accp_ref, wpred_ref, bpred_ref, out_ref):
  g = jnp.concatenate([accp_ref[0], accp_ref[1]], axis=1)
  out_ref[...] = (
      jnp.dot(g, wpred_ref[...], preferred_element_type=jnp.float32)
      + bpred_ref[...]
  )


def _full(shape):
  return pl.BlockSpec(shape, lambda i: tuple(0 for _ in shape))

_GRID = N // BN
_acc_spec = pl.BlockSpec((2, BN, HALF), lambda i: (0, i, 0))
_deg_spec = pl.BlockSpec((2, BN, HALF), lambda i: (0, i, 0))
_self_spec = pl.BlockSpec((BN, DH), lambda i: (i, 0))
_tab_spec = pl.BlockSpec((2, BN, HALF), lambda i: (0, i, 0))

_tab_shape = jax.ShapeDtypeStruct((2, N, HALF), jnp.float32)
_self_shape = jax.ShapeDtypeStruct((N, DH), jnp.float32)

_tc0 = pl.pallas_call(
    _tc0_body,
    grid=(_GRID,),
    in_specs=[
        pl.BlockSpec((BN, D_IN), lambda i: (i, 0)),
        _deg_spec,
        _full((D_IN, DH)),
        _full((1, DH)),
        _full((DH, DH)),
    ],
    out_specs=[_tab_spec, _self_spec],
    out_shape=[_tab_shape, _self_shape],
)

_tc_mid = pl.pallas_call(
    _tc_mid_body,
    grid=(_GRID,),
    in_specs=[_acc_spec, _self_spec, _deg_spec, _full((1, DH)),
              _full((DH, DH))],
    out_specs=[_tab_spec, _self_spec],
    out_shape=[_tab_shape, _self_shape],
)

_tc_emb = pl.pallas_call(
    _tc_emb_body,
    grid=(_GRID,),
    in_specs=[_acc_spec, _self_spec, _deg_spec, _full((1, DH)),
              _full((DH, DH)), _full((1, DH))],
    out_specs=[_tab_spec],
    out_shape=[_tab_shape],
)

_tc_pred = pl.pallas_call(
    _tc_pred_body,
    grid=(1,),
    in_specs=[
        pl.BlockSpec((2, NG, HALF), lambda i: (0, 0, 0)),
        _full((DH, 1)),
        _full((1, 1)),
    ],
    out_specs=[pl.BlockSpec((NG, 1), lambda i: (0, 0))],
    out_shape=[jax.ShapeDtypeStruct((NG, 1), jnp.float32)],
)


def kernel(x, edge_index, batch, W_lin, b_lin, W1, b1, W2, b2, W3, b3,
           W_emb, b_emb, W_pred, b_pred):
  src = edge_index[0]
  dst = edge_index[1]
  i32 = jnp.int32

  pad = EPC - E
  src_c = jnp.concatenate([src, jnp.arange(pad, dtype=i32) % N])
  dst_c = jnp.concatenate([dst, N + jnp.arange(pad, dtype=i32) % NDUM])
  pad = EPD - E
  dst_d = jnp.concatenate([dst, N + jnp.arange(pad, dtype=i32) % NDUM])
  pad = EPP - N
  src_p = jnp.concatenate(
      [jnp.arange(N, dtype=i32), jnp.arange(pad, dtype=i32) % N])
  dst_p = jnp.concatenate(
      [batch, NG + jnp.arange(pad, dtype=i32) % (NPOOL - NG)])

  b_lin2 = b_lin.reshape(1, DH)
  b1_2, b2_2, b3_2 = b1.reshape(1, DH), b2.reshape(1, DH), b3.reshape(1, DH)
  b_emb2 = b_emb.reshape(1, DH)
  b_pred2 = b_pred.reshape(1, 1)

  deg2 = _deg_sc(dst_d).reshape(2, NP, HALF)
  tab1, self1 = _tc0(x, deg2, W_lin, b_lin2, W1)
  acc1 = _conv_sc(tab1.reshape(2 * N, HALF), src_c, dst_c).reshape(2, NP, HALF)
  tab2, self2 = _tc_mid(acc1, self1, deg2, b1_2, W2)
  acc2 = _conv_sc(tab2.reshape(2 * N, HALF), src_c, dst_c).reshape(2, NP, HALF)
  tab3, self3 = _tc_mid(acc2, self2, deg2, b2_2, W3)
  acc3 = _conv_sc(tab3.reshape(2 * N, HALF), src_c, dst_c).reshape(2, NP, HALF)
  tabe, = _tc_emb(acc3, self3, deg2, b3_2, W_emb, b_emb2)
  accp = _pool_sc(tabe.reshape(2 * N, HALF), src_p, dst_p)
  accp = accp.reshape(2, NPOOL, HALF)
  out, = _tc_pred(accp, W_pred, b_pred2)
  return out
